# baseline (speedup 1.0000x reference)
import functools

import jax
import jax.numpy as jnp
from jax import lax
from jax.experimental import pallas as pl
from jax.experimental.pallas import tpu as pltpu
from jax.experimental.pallas import tpu_sc as plsc

N = 10000
E = 320000
D_IN = 128
D_H = 128
N_CLASSES = 40
DC = 128

NP = 10240
NC, NS = 2, 16
NW = NC * NS
EPC = E // NW
CH = 80
NCHUNK = EPC // CH
RPT = NP // NS
WCH = 80
NWCH = RPT // WCH

_mesh = plsc.VectorSubcoreMesh(core_axis_name="c", subcore_axis_name="s")


def _deg_body(ei_hbm, d0s_hbm, d1s_hbm, d0d_hbm, d1d_hbm,
              dsrc_sh, ddst_sh, stg, sidx, didx, ones_v, zv, sem):
  c = lax.axis_index("c")
  s = lax.axis_index("s")
  for i in range(RPT // 16):
    zv[pl.ds(i * 16, 16)] = jnp.zeros((16,), jnp.float32)
  for i in range(CH // 16):
    ones_v[pl.ds(i * 16, 16)] = jnp.full((16,), 1.0, jnp.float32)
  sl = pl.ds(s * RPT, RPT)
  pltpu.sync_copy(zv, dsrc_sh.at[sl])
  pltpu.sync_copy(zv, ddst_sh.at[sl])

  tb = c * (E // 2) + s * EPC

  def rpk(dst2):
    def body(k, carry):
      for j in range(CH // 16):
        dst2[k, pl.ds(j * 16, 16)] = stg[pl.ds(k * CH + j * 16, 16)]
      return carry
    lax.fori_loop(0, NCHUNK, body, 0)

  pltpu.sync_copy(ei_hbm.at[pl.ds(tb, EPC)], stg)
  rpk(sidx)
  pltpu.sync_copy(ei_hbm.at[pl.ds(E + tb, EPC)], stg)
  rpk(didx)
  plsc.subcore_barrier()

  GR = 8
  ngroups = NCHUNK // GR

  def issue(g):
    for b in range(GR):
      k = g * GR + b
      pltpu.async_copy(ones_v, dsrc_sh.at[sidx.at[k]], sem, add=True)
      pltpu.async_copy(ones_v, ddst_sh.at[didx.at[k]], sem, add=True)

  issue(0)

  def grp(g, carry):
    @pl.when(g + 1 < ngroups)
    def _():
      issue(g + 1)

    for _ in range(2 * GR):
      pltpu.make_async_copy(ones_v, dsrc_sh.at[sidx.at[0]], sem).wait()
    return carry

  lax.fori_loop(0, ngroups, grp, 0)
  for t in range(NCHUNK % GR):
    k = (NCHUNK // GR) * GR + t
    pltpu.sync_copy(ones_v, dsrc_sh.at[sidx.at[k]], add=True)
    pltpu.sync_copy(ones_v, ddst_sh.at[didx.at[k]], add=True)
  plsc.subcore_barrier()

  @pl.when(c == 0)
  def _():
    pltpu.sync_copy(dsrc_sh.at[sl], zv)
    pltpu.sync_copy(zv, d0s_hbm.at[sl])
    pltpu.sync_copy(ddst_sh.at[sl], zv)
    pltpu.sync_copy(zv, d0d_hbm.at[sl])

  @pl.when(c == 1)
  def _():
    pltpu.sync_copy(dsrc_sh.at[sl], zv)
    pltpu.sync_copy(zv, d1s_hbm.at[sl])
    pltpu.sync_copy(ddst_sh.at[sl], zv)
    pltpu.sync_copy(zv, d1d_hbm.at[sl])


_deg_kernel = functools.partial(
    pl.kernel,
    out_type=[jax.ShapeDtypeStruct((NP,), jnp.float32)] * 4,
    mesh=_mesh,
    scratch_types=[
        pltpu.VMEM_SHARED((NP,), jnp.float32),
        pltpu.VMEM_SHARED((NP,), jnp.float32),
        pltpu.VMEM((EPC,), jnp.int32),
        pltpu.VMEM((NCHUNK, CH), jnp.int32),
        pltpu.VMEM((NCHUNK, CH), jnp.int32),
        pltpu.VMEM((CH,), jnp.float32),
        pltpu.VMEM((RPT,), jnp.float32),
        pltpu.SemaphoreType.DMA,
    ],
)(_deg_body)


def _make_agg(d):
  def body(h_hbm, ei_hbm, out_hbm, agg_sh, sidx, didx, rb0, rb1,
           sm0, sm1, ss0, ss1):
    c = lax.axis_index("c")
    s = lax.axis_index("s")
    bufs = [rb0, rb1]
    sems = [sm0, sm1]
    ssems = [ss0, ss1]

    def zrow(i, carry):
      for j in range(d // 16):
        rb0[i, pl.ds(j * 16, 16)] = jnp.zeros((16,), jnp.float32)
      return carry

    lax.fori_loop(0, WCH, zrow, 0)
    for j in range(NWCH):
      pltpu.async_copy(rb0, agg_sh.at[pl.ds(s * RPT + j * WCH, WCH)], sm0)

    tb = c * (E // 2) + s * EPC
    pltpu.sync_copy(ei_hbm.at[pl.ds(E + tb, EPC)], sidx)

    def rpk(k, carry):
      for j in range(CH // 16):
        didx[k, pl.ds(j * 16, 16)] = sidx[pl.ds(k * CH + j * 16, 16)]
      return carry

    lax.fori_loop(0, NCHUNK, rpk, 0)
    pltpu.sync_copy(ei_hbm.at[pl.ds(tb, EPC)], sidx)
    for j in range(NWCH):
      pltpu.make_async_copy(rb0, agg_sh.at[pl.ds(s * RPT, WCH)], sm0).wait()

    pltpu.async_copy(h_hbm.at[sidx.at[pl.ds(0, CH)]], bufs[0], sems[0])
    plsc.subcore_barrier()

    ngroups = NCHUNK // 2

    def grp(g, carry):
      for b in range(2):
        k = g * 2 + b
        pltpu.make_async_copy(h_hbm.at[sidx.at[pl.ds(k * CH, CH)]],
                              bufs[b], sems[b]).wait()
        pltpu.async_copy(bufs[b], agg_sh.at[didx.at[k]], ssems[b], add=True)
        if b == 1:
          pltpu.make_async_copy(bufs[0], agg_sh.at[didx.at[0]],
                                ssems[0]).wait()
        else:
          @pl.when(g > 0)
          def _():
            pltpu.make_async_copy(bufs[1], agg_sh.at[didx.at[0]],
                                  ssems[1]).wait()
        pltpu.async_copy(h_hbm.at[sidx.at[pl.ds((k + 1) * CH, CH)]],
                         bufs[1 - b], sems[1 - b])
      return carry

    lax.fori_loop(0, ngroups, grp, 0)
    k = (NCHUNK // 2) * 2
    pltpu.make_async_copy(bufs[1], agg_sh.at[didx.at[0]], ssems[1]).wait()
    pltpu.make_async_copy(h_hbm.at[sidx.at[pl.ds(k * CH, CH)]],
                          bufs[0], sems[0]).wait()
    pltpu.sync_copy(bufs[0], agg_sh.at[didx.at[k]], add=True)
    plsc.subcore_barrier()

    for b in range(2):
      pltpu.async_copy(agg_sh.at[pl.ds(s * RPT + b * WCH, WCH)],
                       bufs[b], sems[b])
    for j in range(NWCH):
      b = j % 2
      pltpu.make_async_copy(agg_sh.at[pl.ds(s * RPT, WCH)],
                            bufs[b], sems[b]).wait()
      pltpu.sync_copy(bufs[b], out_hbm.at[c, pl.ds(s * RPT + j * WCH, WCH)])
      if j + 2 < NWCH:
        pltpu.async_copy(agg_sh.at[pl.ds(s * RPT + (j + 2) * WCH, WCH)],
                         bufs[b], sems[b])

  return functools.partial(
      pl.kernel,
      out_type=jax.ShapeDtypeStruct((2, NP, d), jnp.float32),
      mesh=_mesh,
      scratch_types=[
          pltpu.VMEM_SHARED((NP, d), jnp.float32),
          pltpu.VMEM((EPC,), jnp.int32),
          pltpu.VMEM((NCHUNK, CH), jnp.int32),
          pltpu.VMEM((CH, d), jnp.float32),
          pltpu.VMEM((CH, d), jnp.float32),
          pltpu.SemaphoreType.DMA,
          pltpu.SemaphoreType.DMA,
          pltpu.SemaphoreType.DMA,
          pltpu.SemaphoreType.DMA,
      ],
  )(body)


_agg128 = _make_agg(D_H)


_BT = 2560


def _tc1_body(x_ref, w_ref, dg_ref, h_ref, ns_ref, nd_ref):
  d = dg_ref[...]
  ns = lax.rsqrt(jnp.maximum(d[:, 0:1] + d[:, 1:2], 1.0))
  nd = lax.rsqrt(jnp.maximum(d[:, 2:3] + d[:, 3:4], 1.0))
  h = jnp.dot(x_ref[...], w_ref[...], preferred_element_type=jnp.float32)
  h_ref[...] = h * ns
  ns_ref[...] = ns
  nd_ref[...] = nd


def _tc1(x, w, degt):
  return pl.pallas_call(
      _tc1_body,
      grid=(NP // _BT,),
      in_specs=[
          pl.BlockSpec((_BT, D_IN), lambda i: (i, 0)),
          pl.BlockSpec((D_IN, D_H), lambda i: (0, 0)),
          pl.BlockSpec((_BT, 4), lambda i: (i, 0)),
      ],
      out_specs=[
          pl.BlockSpec((_BT, D_H), lambda i: (i, 0)),
          pl.BlockSpec((_BT, 1), lambda i: (i, 0)),
          pl.BlockSpec((_BT, 1), lambda i: (i, 0)),
      ],
      out_shape=[
          jax.ShapeDtypeStruct((NP, D_H), jnp.float32),
          jax.ShapeDtypeStruct((NP, 1), jnp.float32),
          jax.ShapeDtypeStruct((NP, 1), jnp.float32),
      ],
  )(x, w, degt)


def _mid_body(a0_ref, a1_ref, ns_ref, nd_ref, b_ref, w_ref, h_ref):
  act = (a0_ref[0] + a1_ref[0]) * nd_ref[...] + b_ref[0:1, :]
  act = jnp.maximum(act, 0.0)
  h = jnp.dot(act, w_ref[...], preferred_element_type=jnp.float32)
  h_ref[...] = h * ns_ref[...]


def _tc_mid(p, ns, nd, b8, w, d_out):
  d_in = p.shape[2]
  return pl.pallas_call(
      _mid_body,
      grid=(NP // _BT,),
      in_specs=[
          pl.BlockSpec((1, _BT, d_in), lambda i: (0, i, 0)),
          pl.BlockSpec((1, _BT, d_in), lambda i: (1, i, 0)),
          pl.BlockSpec((_BT, 1), lambda i: (i, 0)),
          pl.BlockSpec((_BT, 1), lambda i: (i, 0)),
          pl.BlockSpec((8, d_in), lambda i: (0, 0)),
          pl.BlockSpec((d_in, d_out), lambda i: (0, 0)),
      ],
      out_specs=pl.BlockSpec((_BT, d_out), lambda i: (i, 0)),
      out_shape=jax.ShapeDtypeStruct((NP, d_out), jnp.float32),
  )(p, p, ns, nd, b8, w)


def _fin_body(a0_ref, a1_ref, nd_ref, b_ref, o_ref):
  o = (a0_ref[0] + a1_ref[0]) * nd_ref[...] + b_ref[0:1, :]
  o_ref[...] = o[:, :N_CLASSES]


def _tc_fin(p, nd, b8):
  return pl.pallas_call(
      _fin_body,
      grid=(NP // _BT,),
      in_specs=[
          pl.BlockSpec((1, _BT, DC), lambda i: (0, i, 0)),
          pl.BlockSpec((1, _BT, DC), lambda i: (1, i, 0)),
          pl.BlockSpec((_BT, 1), lambda i: (i, 0)),
          pl.BlockSpec((8, DC), lambda i: (0, 0)),
      ],
      out_specs=pl.BlockSpec((_BT, N_CLASSES), lambda i: (i, 0)),
      out_shape=jax.ShapeDtypeStruct((NP, N_CLASSES), jnp.float32),
  )(p, p, nd, b8)


def kernel(features, edge_index, W0, b0, W1, b1, W2, b2):
  f32 = jnp.float32
  xp = jnp.zeros((NP, D_IN), f32).at[:N].set(features)
  w2p = jnp.zeros((D_H, DC), f32).at[:, :N_CLASSES].set(W2)
  b0t = jnp.broadcast_to(b0[None, :], (8, D_H))
  b1t = jnp.broadcast_to(b1[None, :], (8, D_H))
  b2t = jnp.zeros((8, DC), f32).at[:, :N_CLASSES].set(
      jnp.broadcast_to(b2[None, :], (8, N_CLASSES)))

  ei1 = edge_index.reshape(2 * E)
  d0s, d1s, d0d, d1d = _deg_kernel(ei1)
  degt = jnp.stack([d0s, d1s, d0d, d1d], axis=1)

  h0, ns, nd = _tc1(xp, W0, degt)
  p1 = _agg128(h0, ei1)
  h1 = _tc_mid(p1, ns, nd, b0t, W1, D_H)
  p2 = _agg128(h1, ei1)
  h2 = _tc_mid(p2, ns, nd, b1t, w2p, DC)
  p3 = _agg128(h2, ei1)
  logits = _tc_fin(p3, nd, b2t)
  return logits[:N]


# speedup vs baseline: 1.2534x; 1.2534x over previous
import functools

import jax
import jax.numpy as jnp
from jax import lax
from jax.experimental import pallas as pl
from jax.experimental.pallas import tpu as pltpu
from jax.experimental.pallas import tpu_sc as plsc

N = 10000
E = 320000
D_IN = 128
D_H = 128
N_CLASSES = 40
DC = 128

NP = 10240
NC, NS = 2, 16
NW = NC * NS
EPC = E // NW
CH = 80
NCHUNK = EPC // CH
RPT = NP // NS
WCH = 80
NWCH = RPT // WCH

_mesh = plsc.VectorSubcoreMesh(core_axis_name="c", subcore_axis_name="s")


def _deg_body(ei_hbm, d0s_hbm, d1s_hbm, d0d_hbm, d1d_hbm,
              dsrc_sh, ddst_sh, stg, sidx, didx, ones_v, zv, sem):
  c = lax.axis_index("c")
  s = lax.axis_index("s")
  for i in range(RPT // 16):
    zv[pl.ds(i * 16, 16)] = jnp.zeros((16,), jnp.float32)
  for i in range(CH // 16):
    ones_v[pl.ds(i * 16, 16)] = jnp.full((16,), 1.0, jnp.float32)
  sl = pl.ds(s * RPT, RPT)
  pltpu.sync_copy(zv, dsrc_sh.at[sl])
  pltpu.sync_copy(zv, ddst_sh.at[sl])

  tb = c * (E // 2) + s * EPC

  def rpk(dst2):
    def body(k, carry):
      for j in range(CH // 16):
        dst2[k, pl.ds(j * 16, 16)] = stg[pl.ds(k * CH + j * 16, 16)]
      return carry
    lax.fori_loop(0, NCHUNK, body, 0)

  pltpu.sync_copy(ei_hbm.at[pl.ds(tb, EPC)], stg)
  rpk(sidx)
  pltpu.sync_copy(ei_hbm.at[pl.ds(E + tb, EPC)], stg)
  rpk(didx)
  plsc.subcore_barrier()

  GR = 8
  ngroups = NCHUNK // GR

  def issue(g):
    for b in range(GR):
      k = g * GR + b
      pltpu.async_copy(ones_v, dsrc_sh.at[sidx.at[k]], sem, add=True)
      pltpu.async_copy(ones_v, ddst_sh.at[didx.at[k]], sem, add=True)

  issue(0)

  def grp(g, carry):
    @pl.when(g + 1 < ngroups)
    def _():
      issue(g + 1)

    for _ in range(2 * GR):
      pltpu.make_async_copy(ones_v, dsrc_sh.at[sidx.at[0]], sem).wait()
    return carry

  lax.fori_loop(0, ngroups, grp, 0)
  for t in range(NCHUNK % GR):
    k = (NCHUNK // GR) * GR + t
    pltpu.sync_copy(ones_v, dsrc_sh.at[sidx.at[k]], add=True)
    pltpu.sync_copy(ones_v, ddst_sh.at[didx.at[k]], add=True)
  plsc.subcore_barrier()

  @pl.when(c == 0)
  def _():
    pltpu.sync_copy(dsrc_sh.at[sl], zv)
    pltpu.sync_copy(zv, d0s_hbm.at[sl])
    pltpu.sync_copy(ddst_sh.at[sl], zv)
    pltpu.sync_copy(zv, d0d_hbm.at[sl])

  @pl.when(c == 1)
  def _():
    pltpu.sync_copy(dsrc_sh.at[sl], zv)
    pltpu.sync_copy(zv, d1s_hbm.at[sl])
    pltpu.sync_copy(ddst_sh.at[sl], zv)
    pltpu.sync_copy(zv, d1d_hbm.at[sl])


_deg_kernel = functools.partial(
    pl.kernel,
    out_type=[jax.ShapeDtypeStruct((NP,), jnp.float32)] * 4,
    mesh=_mesh,
    scratch_types=[
        pltpu.VMEM_SHARED((NP,), jnp.float32),
        pltpu.VMEM_SHARED((NP,), jnp.float32),
        pltpu.VMEM((EPC,), jnp.int32),
        pltpu.VMEM((NCHUNK, CH), jnp.int32),
        pltpu.VMEM((NCHUNK, CH), jnp.int32),
        pltpu.VMEM((CH,), jnp.float32),
        pltpu.VMEM((RPT,), jnp.float32),
        pltpu.SemaphoreType.DMA,
    ],
)(_deg_body)


def _make_agg(d):
  def body(h_hbm, ei_hbm, out_hbm, agg_sh, sidx, didx, rb0, rb1, sm0, sm1):
    c = lax.axis_index("c")
    s = lax.axis_index("s")
    bufs = [rb0, rb1]
    sems = [sm0, sm1]

    def zrow(i, carry):
      for j in range(d // 16):
        rb0[i, pl.ds(j * 16, 16)] = jnp.zeros((16,), jnp.float32)
      return carry

    lax.fori_loop(0, WCH, zrow, 0)
    for j in range(NWCH):
      pltpu.async_copy(rb0, agg_sh.at[pl.ds(s * RPT + j * WCH, WCH)], sm0)

    tb = c * (E // 2) + s * EPC
    pltpu.sync_copy(ei_hbm.at[pl.ds(E + tb, EPC)], sidx)

    def rpk(k, carry):
      for j in range(CH // 16):
        didx[k, pl.ds(j * 16, 16)] = sidx[pl.ds(k * CH + j * 16, 16)]
      return carry

    lax.fori_loop(0, NCHUNK, rpk, 0)
    pltpu.sync_copy(ei_hbm.at[pl.ds(tb, EPC)], sidx)
    for j in range(NWCH):
      pltpu.make_async_copy(rb0, agg_sh.at[pl.ds(s * RPT, WCH)], sm0).wait()

    for b in range(2):
      pltpu.async_copy(h_hbm.at[sidx.at[pl.ds(b * CH, CH)]], bufs[b], sems[b])
    plsc.subcore_barrier()

    ngroups = NCHUNK // 2

    def grp(g, carry):
      for b in range(2):
        k = g * 2 + b
        pltpu.make_async_copy(h_hbm.at[sidx.at[pl.ds(k * CH, CH)]],
                              bufs[b], sems[b]).wait()
        pltpu.sync_copy(bufs[b], agg_sh.at[didx.at[k]], add=True)

        @pl.when(g + 1 < ngroups)
        def _():
          pltpu.async_copy(h_hbm.at[sidx.at[pl.ds((k + 2) * CH, CH)]],
                           bufs[b], sems[b])

      return carry

    lax.fori_loop(0, ngroups, grp, 0)
    for t in range(NCHUNK % 2):
      k = (NCHUNK // 2) * 2 + t
      pltpu.async_copy(h_hbm.at[sidx.at[pl.ds(k * CH, CH)]], bufs[t],
                       sems[t]).wait()
      pltpu.sync_copy(bufs[t], agg_sh.at[didx.at[k]], add=True)
    plsc.subcore_barrier()

    for b in range(2):
      pltpu.async_copy(agg_sh.at[pl.ds(s * RPT + b * WCH, WCH)],
                       bufs[b], sems[b])
    for j in range(NWCH):
      b = j % 2
      pltpu.make_async_copy(agg_sh.at[pl.ds(s * RPT, WCH)],
                            bufs[b], sems[b]).wait()
      pltpu.sync_copy(bufs[b], out_hbm.at[c, pl.ds(s * RPT + j * WCH, WCH)])
      if j + 2 < NWCH:
        pltpu.async_copy(agg_sh.at[pl.ds(s * RPT + (j + 2) * WCH, WCH)],
                         bufs[b], sems[b])

  return functools.partial(
      pl.kernel,
      out_type=jax.ShapeDtypeStruct((2, NP, d), jnp.float32),
      mesh=_mesh,
      scratch_types=[
          pltpu.VMEM_SHARED((NP, d), jnp.float32),
          pltpu.VMEM((EPC,), jnp.int32),
          pltpu.VMEM((NCHUNK, CH), jnp.int32),
          pltpu.VMEM((CH, d), jnp.float32),
          pltpu.VMEM((CH, d), jnp.float32),
          pltpu.SemaphoreType.DMA,
          pltpu.SemaphoreType.DMA,
      ],
  )(body)


_agg128 = _make_agg(D_H)


_BT = 2560


def _tca_body(x_ref, w_ref, h_ref):
  h_ref[...] = jnp.dot(x_ref[...], w_ref[...],
                       preferred_element_type=jnp.float32)


def _tca(x, w):
  return pl.pallas_call(
      _tca_body,
      grid=(NP // _BT,),
      in_specs=[
          pl.BlockSpec((_BT, D_IN), lambda i: (i, 0)),
          pl.BlockSpec((D_IN, D_H), lambda i: (0, 0)),
      ],
      out_specs=pl.BlockSpec((_BT, D_H), lambda i: (i, 0)),
      out_shape=jax.ShapeDtypeStruct((NP, D_H), jnp.float32),
  )(x, w)


def _tcb_body(hu_ref, dg_ref, h_ref, ns_ref, nd_ref):
  d = dg_ref[...]
  ns = lax.rsqrt(jnp.maximum(d[:, 0:1] + d[:, 1:2], 1.0))
  nd = lax.rsqrt(jnp.maximum(d[:, 2:3] + d[:, 3:4], 1.0))
  h_ref[...] = hu_ref[...] * ns
  ns_ref[...] = ns
  nd_ref[...] = nd


def _tcb(hu, degt):
  return pl.pallas_call(
      _tcb_body,
      grid=(NP // _BT,),
      in_specs=[
          pl.BlockSpec((_BT, D_H), lambda i: (i, 0)),
          pl.BlockSpec((_BT, 4), lambda i: (i, 0)),
      ],
      out_specs=[
          pl.BlockSpec((_BT, D_H), lambda i: (i, 0)),
          pl.BlockSpec((_BT, 1), lambda i: (i, 0)),
          pl.BlockSpec((_BT, 1), lambda i: (i, 0)),
      ],
      out_shape=[
          jax.ShapeDtypeStruct((NP, D_H), jnp.float32),
          jax.ShapeDtypeStruct((NP, 1), jnp.float32),
          jax.ShapeDtypeStruct((NP, 1), jnp.float32),
      ],
  )(hu, degt)


def _mid_body(a0_ref, a1_ref, ns_ref, nd_ref, b_ref, w_ref, h_ref):
  act = (a0_ref[0] + a1_ref[0]) * nd_ref[...] + b_ref[0:1, :]
  act = jnp.maximum(act, 0.0)
  h = jnp.dot(act, w_ref[...], preferred_element_type=jnp.float32)
  h_ref[...] = h * ns_ref[...]


def _tc_mid(p, ns, nd, b8, w, d_out):
  d_in = p.shape[2]
  return pl.pallas_call(
      _mid_body,
      grid=(NP // _BT,),
      in_specs=[
          pl.BlockSpec((1, _BT, d_in), lambda i: (0, i, 0)),
          pl.BlockSpec((1, _BT, d_in), lambda i: (1, i, 0)),
          pl.BlockSpec((_BT, 1), lambda i: (i, 0)),
          pl.BlockSpec((_BT, 1), lambda i: (i, 0)),
          pl.BlockSpec((8, d_in), lambda i: (0, 0)),
          pl.BlockSpec((d_in, d_out), lambda i: (0, 0)),
      ],
      out_specs=pl.BlockSpec((_BT, d_out), lambda i: (i, 0)),
      out_shape=jax.ShapeDtypeStruct((NP, d_out), jnp.float32),
  )(p, p, ns, nd, b8, w)


def _fin_body(a0_ref, a1_ref, nd_ref, b_ref, o_ref):
  o = (a0_ref[0] + a1_ref[0]) * nd_ref[...] + b_ref[0:1, :]
  o_ref[...] = o[:, :N_CLASSES]


def _tc_fin(p, nd, b8):
  return pl.pallas_call(
      _fin_body,
      grid=(NP // _BT,),
      in_specs=[
          pl.BlockSpec((1, _BT, DC), lambda i: (0, i, 0)),
          pl.BlockSpec((1, _BT, DC), lambda i: (1, i, 0)),
          pl.BlockSpec((_BT, 1), lambda i: (i, 0)),
          pl.BlockSpec((8, DC), lambda i: (0, 0)),
      ],
      out_specs=pl.BlockSpec((_BT, N_CLASSES), lambda i: (i, 0)),
      out_shape=jax.ShapeDtypeStruct((NP, N_CLASSES), jnp.float32),
  )(p, p, nd, b8)


def kernel(features, edge_index, W0, b0, W1, b1, W2, b2):
  f32 = jnp.float32
  xp = jnp.zeros((NP, D_IN), f32).at[:N].set(features)
  w2p = jnp.zeros((D_H, DC), f32).at[:, :N_CLASSES].set(W2)
  b0t = jnp.broadcast_to(b0[None, :], (8, D_H))
  b1t = jnp.broadcast_to(b1[None, :], (8, D_H))
  b2t = jnp.zeros((8, DC), f32).at[:, :N_CLASSES].set(
      jnp.broadcast_to(b2[None, :], (8, N_CLASSES)))

  ei1 = edge_index.reshape(2 * E)
  d0s, d1s, d0d, d1d = _deg_kernel(ei1)
  degt = jnp.stack([d0s, d1s, d0d, d1d], axis=1)

  h0u = _tca(xp, W0)
  h0, ns, nd = _tcb(h0u, degt)
  p1 = _agg128(h0, ei1)
  h1 = _tc_mid(p1, ns, nd, b0t, W1, D_H)
  p2 = _agg128(h1, ei1)
  h2 = _tc_mid(p2, ns, nd, b1t, w2p, DC)
  p3 = _agg128(h2, ei1)
  logits = _tc_fin(p3, nd, b2t)
  return logits[:N]


# speedup vs baseline: 1.2555x; 1.0016x over previous
import functools

import jax
import jax.numpy as jnp
from jax import lax
from jax.experimental import pallas as pl
from jax.experimental.pallas import tpu as pltpu
from jax.experimental.pallas import tpu_sc as plsc

N = 10000
E = 320000
D_IN = 128
D_H = 128
N_CLASSES = 40
DC = 128

NP = 10240
NC, NS = 2, 16
NW = NC * NS
EPC = E // NW
CH = 80
NCHUNK = EPC // CH
RPT = NP // NS
WCH = 80
NWCH = RPT // WCH

_mesh = plsc.VectorSubcoreMesh(core_axis_name="c", subcore_axis_name="s")


def _deg_body(ei_hbm, d0s_hbm, d1s_hbm, d0d_hbm, d1d_hbm,
              dsrc_sh, ddst_sh, stg, sidx, didx, ones_v, zv, sem):
  c = lax.axis_index("c")
  s = lax.axis_index("s")
  for i in range(RPT // 16):
    zv[pl.ds(i * 16, 16)] = jnp.zeros((16,), jnp.float32)
  for i in range(CH // 16):
    ones_v[pl.ds(i * 16, 16)] = jnp.full((16,), 1.0, jnp.float32)
  sl = pl.ds(s * RPT, RPT)
  pltpu.sync_copy(zv, dsrc_sh.at[sl])
  pltpu.sync_copy(zv, ddst_sh.at[sl])

  tb = c * (E // 2) + s * EPC

  def rpk(dst2):
    def body(k, carry):
      for j in range(CH // 16):
        dst2[k, pl.ds(j * 16, 16)] = stg[pl.ds(k * CH + j * 16, 16)]
      return carry
    lax.fori_loop(0, NCHUNK, body, 0)

  pltpu.sync_copy(ei_hbm.at[pl.ds(tb, EPC)], stg)
  rpk(sidx)
  pltpu.sync_copy(ei_hbm.at[pl.ds(E + tb, EPC)], stg)
  rpk(didx)
  plsc.subcore_barrier()

  GR = 8
  ngroups = NCHUNK // GR

  def issue(g):
    for b in range(GR):
      k = g * GR + b
      pltpu.async_copy(ones_v, dsrc_sh.at[sidx.at[k]], sem, add=True)
      pltpu.async_copy(ones_v, ddst_sh.at[didx.at[k]], sem, add=True)

  issue(0)

  def grp(g, carry):
    @pl.when(g + 1 < ngroups)
    def _():
      issue(g + 1)

    for _ in range(2 * GR):
      pltpu.make_async_copy(ones_v, dsrc_sh.at[sidx.at[0]], sem).wait()
    return carry

  lax.fori_loop(0, ngroups, grp, 0)
  for t in range(NCHUNK % GR):
    k = (NCHUNK // GR) * GR + t
    pltpu.sync_copy(ones_v, dsrc_sh.at[sidx.at[k]], add=True)
    pltpu.sync_copy(ones_v, ddst_sh.at[didx.at[k]], add=True)
  plsc.subcore_barrier()

  @pl.when(c == 0)
  def _():
    pltpu.sync_copy(dsrc_sh.at[sl], zv)
    pltpu.sync_copy(zv, d0s_hbm.at[sl])
    pltpu.sync_copy(ddst_sh.at[sl], zv)
    pltpu.sync_copy(zv, d0d_hbm.at[sl])

  @pl.when(c == 1)
  def _():
    pltpu.sync_copy(dsrc_sh.at[sl], zv)
    pltpu.sync_copy(zv, d1s_hbm.at[sl])
    pltpu.sync_copy(ddst_sh.at[sl], zv)
    pltpu.sync_copy(zv, d1d_hbm.at[sl])


_deg_kernel = functools.partial(
    pl.kernel,
    out_type=[jax.ShapeDtypeStruct((NP,), jnp.float32)] * 4,
    mesh=_mesh,
    scratch_types=[
        pltpu.VMEM_SHARED((NP,), jnp.float32),
        pltpu.VMEM_SHARED((NP,), jnp.float32),
        pltpu.VMEM((EPC,), jnp.int32),
        pltpu.VMEM((NCHUNK, CH), jnp.int32),
        pltpu.VMEM((NCHUNK, CH), jnp.int32),
        pltpu.VMEM((CH,), jnp.float32),
        pltpu.VMEM((RPT,), jnp.float32),
        pltpu.SemaphoreType.DMA,
    ],
)(_deg_body)


def _make_agg(d):
  def body(h_hbm, ei_hbm, out_hbm, agg_sh, sidx, didx, rb0, rb1, sm0, sm1):
    c = lax.axis_index("c")
    s = lax.axis_index("s")
    bufs = [rb0, rb1]
    sems = [sm0, sm1]

    def zrow(i, carry):
      for j in range(d // 16):
        rb0[i, pl.ds(j * 16, 16)] = jnp.zeros((16,), jnp.float32)
      return carry

    lax.fori_loop(0, WCH, zrow, 0)
    for j in range(NWCH):
      pltpu.async_copy(rb0, agg_sh.at[pl.ds(s * RPT + j * WCH, WCH)], sm0)

    tb = c * (E // 2) + s * EPC
    pltpu.sync_copy(ei_hbm.at[pl.ds(E + tb, EPC)], sidx)

    def rpk(k, carry):
      for j in range(CH // 16):
        didx[k, pl.ds(j * 16, 16)] = sidx[pl.ds(k * CH + j * 16, 16)]
      return carry

    lax.fori_loop(0, NCHUNK, rpk, 0)
    pltpu.sync_copy(ei_hbm.at[pl.ds(tb, EPC)], sidx)
    for j in range(NWCH):
      pltpu.make_async_copy(rb0, agg_sh.at[pl.ds(s * RPT, WCH)], sm0).wait()

    for b in range(2):
      pltpu.async_copy(h_hbm.at[sidx.at[pl.ds(b * CH, CH)]], bufs[b], sems[b])
    plsc.subcore_barrier()

    ngroups = NCHUNK // 2

    def grp(g, carry):
      for b in range(2):
        k = g * 2 + b
        pltpu.make_async_copy(h_hbm.at[sidx.at[pl.ds(k * CH, CH)]],
                              bufs[b], sems[b]).wait()
        pltpu.sync_copy(bufs[b], agg_sh.at[didx.at[k]], add=True)

        @pl.when(g + 1 < ngroups)
        def _():
          pltpu.async_copy(h_hbm.at[sidx.at[pl.ds((k + 2) * CH, CH)]],
                           bufs[b], sems[b])

      return carry

    lax.fori_loop(0, ngroups, grp, 0)
    for t in range(NCHUNK % 2):
      k = (NCHUNK // 2) * 2 + t
      pltpu.async_copy(h_hbm.at[sidx.at[pl.ds(k * CH, CH)]], bufs[t],
                       sems[t]).wait()
      pltpu.sync_copy(bufs[t], agg_sh.at[didx.at[k]], add=True)
    plsc.subcore_barrier()

    for b in range(2):
      pltpu.async_copy(agg_sh.at[pl.ds(s * RPT + b * WCH, WCH)],
                       bufs[b], sems[b])
    for j in range(NWCH):
      b = j % 2
      pltpu.make_async_copy(agg_sh.at[pl.ds(s * RPT, WCH)],
                            bufs[b], sems[b]).wait()
      pltpu.sync_copy(bufs[b], out_hbm.at[c, pl.ds(s * RPT + j * WCH, WCH)])
      if j + 2 < NWCH:
        pltpu.async_copy(agg_sh.at[pl.ds(s * RPT + (j + 2) * WCH, WCH)],
                         bufs[b], sems[b])

  return functools.partial(
      pl.kernel,
      out_type=jax.ShapeDtypeStruct((2, NP, d), jnp.float32),
      mesh=_mesh,
      scratch_types=[
          pltpu.VMEM_SHARED((NP, d), jnp.float32),
          pltpu.VMEM((EPC,), jnp.int32),
          pltpu.VMEM((NCHUNK, CH), jnp.int32),
          pltpu.VMEM((CH, d), jnp.float32),
          pltpu.VMEM((CH, d), jnp.float32),
          pltpu.SemaphoreType.DMA,
          pltpu.SemaphoreType.DMA,
      ],
  )(body)


_agg128 = _make_agg(D_H)


_BT = 2560


def _tc1_body(x_ref, w_ref, dg_ref, h_ref, ns_ref, nd_ref):
  d = dg_ref[...]
  ns = lax.rsqrt(jnp.maximum(d[:, 0:1] + d[:, 1:2], 1.0))
  nd = lax.rsqrt(jnp.maximum(d[:, 2:3] + d[:, 3:4], 1.0))
  h = jnp.dot(x_ref[...], w_ref[...], preferred_element_type=jnp.float32)
  h_ref[...] = h * ns
  ns_ref[...] = ns
  nd_ref[...] = nd


def _tc1(x, w, degt):
  return pl.pallas_call(
      _tc1_body,
      grid=(NP // _BT,),
      in_specs=[
          pl.BlockSpec((_BT, D_IN), lambda i: (i, 0)),
          pl.BlockSpec((D_IN, D_H), lambda i: (0, 0)),
          pl.BlockSpec((_BT, 4), lambda i: (i, 0)),
      ],
      out_specs=[
          pl.BlockSpec((_BT, D_H), lambda i: (i, 0)),
          pl.BlockSpec((_BT, 1), lambda i: (i, 0)),
          pl.BlockSpec((_BT, 1), lambda i: (i, 0)),
      ],
      out_shape=[
          jax.ShapeDtypeStruct((NP, D_H), jnp.float32),
          jax.ShapeDtypeStruct((NP, 1), jnp.float32),
          jax.ShapeDtypeStruct((NP, 1), jnp.float32),
      ],
  )(x, w, degt)


def _mid_body(a0_ref, a1_ref, ns_ref, nd_ref, b_ref, w_ref, h_ref):
  act = (a0_ref[0] + a1_ref[0]) * nd_ref[...] + b_ref[0:1, :]
  act = jnp.maximum(act, 0.0)
  h = jnp.dot(act, w_ref[...], preferred_element_type=jnp.float32)
  h_ref[...] = h * ns_ref[...]


def _tc_mid(p, ns, nd, b8, w, d_out):
  d_in = p.shape[2]
  return pl.pallas_call(
      _mid_body,
      grid=(NP // _BT,),
      in_specs=[
          pl.BlockSpec((1, _BT, d_in), lambda i: (0, i, 0)),
          pl.BlockSpec((1, _BT, d_in), lambda i: (1, i, 0)),
          pl.BlockSpec((_BT, 1), lambda i: (i, 0)),
          pl.BlockSpec((_BT, 1), lambda i: (i, 0)),
          pl.BlockSpec((8, d_in), lambda i: (0, 0)),
          pl.BlockSpec((d_in, d_out), lambda i: (0, 0)),
      ],
      out_specs=pl.BlockSpec((_BT, d_out), lambda i: (i, 0)),
      out_shape=jax.ShapeDtypeStruct((NP, d_out), jnp.float32),
  )(p, p, ns, nd, b8, w)


def _fin_body(a0_ref, a1_ref, nd_ref, b_ref, o_ref):
  o = (a0_ref[0] + a1_ref[0]) * nd_ref[...] + b_ref[0:1, :]
  o_ref[...] = o[:, :N_CLASSES]


def _tc_fin(p, nd, b8):
  return pl.pallas_call(
      _fin_body,
      grid=(NP // _BT,),
      in_specs=[
          pl.BlockSpec((1, _BT, DC), lambda i: (0, i, 0)),
          pl.BlockSpec((1, _BT, DC), lambda i: (1, i, 0)),
          pl.BlockSpec((_BT, 1), lambda i: (i, 0)),
          pl.BlockSpec((8, DC), lambda i: (0, 0)),
      ],
      out_specs=pl.BlockSpec((_BT, N_CLASSES), lambda i: (i, 0)),
      out_shape=jax.ShapeDtypeStruct((NP, N_CLASSES), jnp.float32),
  )(p, p, nd, b8)


def kernel(features, edge_index, W0, b0, W1, b1, W2, b2):
  f32 = jnp.float32
  xp = jnp.zeros((NP, D_IN), f32).at[:N].set(features)
  w2p = jnp.zeros((D_H, DC), f32).at[:, :N_CLASSES].set(W2)
  b0t = jnp.broadcast_to(b0[None, :], (8, D_H))
  b1t = jnp.broadcast_to(b1[None, :], (8, D_H))
  b2t = jnp.zeros((8, DC), f32).at[:, :N_CLASSES].set(
      jnp.broadcast_to(b2[None, :], (8, N_CLASSES)))

  ei1 = edge_index.reshape(2 * E)
  d0s, d1s, d0d, d1d = _deg_kernel(ei1)
  degt = jnp.stack([d0s, d1s, d0d, d1d], axis=1)

  h0, ns, nd = _tc1(xp, W0, degt)
  p1 = _agg128(h0, ei1)
  h1 = _tc_mid(p1, ns, nd, b0t, W1, D_H)
  p2 = _agg128(h1, ei1)
  h2 = _tc_mid(p2, ns, nd, b1t, w2p, DC)
  p3 = _agg128(h2, ei1)
  logits = _tc_fin(p3, nd, b2t)
  return logits[:N]
